# baseline (speedup 1.0000x reference)
import dataclasses
import functools

import jax
import jax.numpy as jnp
from jax.experimental import pallas as pl
from jax.experimental.pallas import tpu as pltpu
from jax.experimental.pallas import tpu_sc as plsc

HIDDEN = 128
LANES = 16
SCALE = float(HIDDEN) ** 0.5
WINDOW = 128


def _emb_kernel(n_idx, table, idx):
    mesh = plsc.VectorSubcoreMesh(core_axis_name="core", subcore_axis_name="subcore")

    cp = pltpu.CompilerParams()
    if "needs_layout_passes" in pltpu.CompilerParams.__dataclass_fields__:
        cp = dataclasses.replace(cp, needs_layout_passes=False)

    @functools.partial(
        pl.kernel,
        out_type=jax.ShapeDtypeStruct((n_idx, HIDDEN), jnp.float32),
        mesh=mesh,
        compiler_params=cp,
    )
    def run(table_hbm, idx_hbm, out_hbm):
        def body(i_vmem, o_vmem):
            pltpu.sync_copy(table_hbm.at[i_vmem.at[0]], o_vmem)

            @pl.loop(0, WINDOW)
            def _(r):
                lane_r = jnp.full((LANES,), r, jnp.int32)
                iv = plsc.load_gather(
                    i_vmem, [jnp.zeros((LANES,), jnp.int32), lane_r]
                )
                sv = jnp.where(iv != 0, SCALE, 0.0).astype(jnp.float32)
                for j in range(HIDDEN // LANES):
                    ref = o_vmem.at[r, pl.ds(j * LANES, LANES)]
                    ref[...] = ref[...] * sv

        pltpu.emit_pipeline(
            body,
            grid=(n_idx // WINDOW,),
            in_specs=[pl.BlockSpec((1, WINDOW), lambda i: (0, i))],
            out_specs=[pl.BlockSpec((WINDOW, HIDDEN), lambda i: (i, 0))],
            core_axis_name=("core", "subcore"),
            dimension_semantics=(pltpu.PARALLEL,),
        )(idx_hbm, out_hbm)

    return run(table, idx)


def kernel(x, shared_weights):
    batch, seq = x.shape
    n_idx = batch * seq
    idx = x.reshape(1, n_idx)
    out = _emb_kernel(n_idx, shared_weights, idx)
    return out.reshape(batch, seq, HIDDEN)
